# baseline (speedup 1.0000x reference)
import functools

import jax
import jax.numpy as jnp
from jax import lax
from jax.experimental import pallas as pl
from jax.experimental.pallas import tpu as pltpu
from jax.experimental.pallas import tpu_sc as plsc

N_NODES = 100000
N_EDGES = 3200000
NPAD = 102400
EPS = 1e-6

NB = N_EDGES // 512
BPW = 196
RPT = NPAD // 16
VCH = RPT // 640
FNB = NPAD // 512
FBW = 7

_mesh = plsc.VectorSubcoreMesh(core_axis_name="c", subcore_axis_name="s")

_f32 = jnp.float32
_i32 = jnp.int32


def _rsqrt(x):
    i = lax.bitcast_convert_type(x, _i32)
    i = jnp.int32(0x5F3759DF) - (i >> 1)
    y = lax.bitcast_convert_type(i, _f32)
    y = y * (1.5 - 0.5 * x * y * y)
    y = y * (1.5 - 0.5 * x * y * y)
    return y


def _log(x):
    bits = lax.bitcast_convert_type(x, _i32)
    e = (bits >> 23) - 127
    m = lax.bitcast_convert_type((bits & 0x7FFFFF) | 0x3F800000, _f32)
    big = m > 1.4142135
    m = jnp.where(big, m * 0.5, m)
    ef = (e + big.astype(_i32)).astype(_f32)
    t = (m - 1.0) / (m + 1.0)
    t2 = t * t
    p = 2.0 * t * (1.0 + t2 * (1.0 / 3.0 + t2 * (0.2 + t2 * (1.0 / 7.0))))
    return ef * 0.6931471805599453 + p


def _dg(v, idx):
    return lax.gather(
        v, idx[:, None],
        lax.GatherDimensionNumbers(offset_dims=(), collapsed_slice_dims=(0,),
                                   start_index_map=(0,)),
        (1,), mode=lax.GatherScatterMode.PROMISE_IN_BOUNDS)


_PART = jax.ShapeDtypeStruct((NPAD,), _f32)


def _deint3(vA, vB, vC, c, iota):
    pos = 3 * iota + c
    idx = pos & 15
    sel = pos >> 4
    return jnp.where(sel == 0, _dg(vA, idx),
                     jnp.where(sel == 1, _dg(vB, idx), _dg(vC, idx)))


def _int3(o0, o1, o2, k, iota):
    pos = 16 * k + iota
    idx = (pos * 43691) >> 17
    cmp = pos - 3 * idx
    return jnp.where(cmp == 0, _dg(o0, idx),
                     jnp.where(cmp == 1, _dg(o1, idx), _dg(o2, idx)))


@functools.partial(
    pl.kernel,
    mesh=_mesh,
    compiler_params=pltpu.CompilerParams(needs_layout_passes=False),
    out_type=[_PART] * 11,
    scratch_types=(
        [pltpu.VMEM_SHARED((NPAD,), _f32)] * 7
        + [pltpu.VMEM((1920,), _f32)]
        + [pltpu.VMEM((640,), _f32)] * 3
        + [pltpu.VMEM((1024,), _i32)]
        + [pltpu.VMEM((512,), _i32)] * 2
        + [pltpu.VMEM((2048,), _f32)]
        + [pltpu.VMEM((512,), _f32)] * 10
        + [pltpu.SemaphoreType.DMA] * 2
    ),
)
def _agg(vp_h, edges_h, msg_h, z_h,
         s0t, s01, s02, s0c, s1t, s11, s12, s1c, t0h, t1h, t2h,
         vt, v1, v2, at, a1, a2, ac,
         vfl, tb0, tb1, tb2, ed_v, si, di, msgv,
         xt_v, x1_v, x2_v, yt_v, y1_v, y2_v, ot_v, o1_v, o2_v, ones_v,
         sem, sem2):
    c = lax.axis_index("c")
    s = lax.axis_index("s")
    wid = s * 2 + c
    lo = s * RPT
    iota = lax.iota(_i32, 16)

    for ch in range(VCH):
        rbase = lo + ch * 640
        pltpu.sync_copy(vp_h.at[pl.ds(rbase * 3, 1920)], vfl)
        for g in range(40):
            vA = vfl[pl.ds(g * 48, 16)]
            vB = vfl[pl.ds(g * 48 + 16, 16)]
            vC = vfl[pl.ds(g * 48 + 32, 16)]
            gs = pl.ds(g * 16, 16)
            tb0[gs] = _deint3(vA, vB, vC, 0, iota)
            tb1[gs] = _deint3(vA, vB, vC, 1, iota)
            tb2[gs] = _deint3(vA, vB, vC, 2, iota)
        cs = pl.ds(rbase, 640)
        pltpu.sync_copy(tb0, vt.at[cs])
        pltpu.sync_copy(tb1, v1.at[cs])
        pltpu.sync_copy(tb2, v2.at[cs])

        @pl.when(c == 0)
        def _():
            pltpu.sync_copy(tb0, t0h.at[cs])
            pltpu.sync_copy(tb1, t1h.at[cs])
            pltpu.sync_copy(tb2, t2h.at[cs])

    sl = pl.ds(lo, RPT)
    pltpu.sync_copy(z_h, at.at[sl])
    pltpu.sync_copy(z_h, a1.at[sl])
    pltpu.sync_copy(z_h, a2.at[sl])
    pltpu.sync_copy(z_h, ac.at[sl])
    plsc.subcore_barrier()

    iota = lax.iota(_i32, 16)
    i1 = iota ^ 1
    i2 = iota ^ 2
    pidx = (iota & 3) * 4
    m0 = iota < 4
    m1 = iota < 8
    m2 = iota < 12
    mlo = iota < 8
    e2a = (iota * 2) & 15
    e2b = (iota * 2 + 1) & 15
    one16 = jnp.ones((16,), _f32)
    for g in range(32):
        ones_v[pl.ds(g * 16, 16)] = one16

    def block(i, carry):
        b = wid * BPW + i

        @pl.when(b < NB)
        def _():
            pltpu.sync_copy(edges_h.at[pl.ds(b * 1024, 1024)], ed_v)
            pltpu.sync_copy(msg_h.at[pl.ds(b * 2048, 2048)], msgv)
            for g in range(32):
                vA = ed_v[pl.ds(g * 32, 16)]
                vB = ed_v[pl.ds(g * 32 + 16, 16)]
                gs = pl.ds(g * 16, 16)
                si[gs] = jnp.where(mlo, _dg(vA, e2a), _dg(vB, e2a))
                di[gs] = jnp.where(mlo, _dg(vA, e2b), _dg(vB, e2b))
            cps = [pltpu.async_copy(vt.at[si], xt_v, sem),
                   pltpu.async_copy(v1.at[si], x1_v, sem),
                   pltpu.async_copy(v2.at[si], x2_v, sem),
                   pltpu.async_copy(vt.at[di], yt_v, sem),
                   pltpu.async_copy(v1.at[di], y1_v, sem),
                   pltpu.async_copy(v2.at[di], y2_v, sem)]
            for cp in cps:
                cp.wait()
            for g in range(32):
                gs = pl.ds(g * 16, 16)
                us = []
                for k in range(4):
                    v = msgv[pl.ds(g * 64 + k * 16, 16)]
                    u = v + _dg(v, i1)
                    us.append(u + _dg(u, i2))
                w = jnp.where(
                    m0, _dg(us[0], pidx),
                    jnp.where(m1, _dg(us[1], pidx),
                              jnp.where(m2, _dg(us[2], pidx),
                                        _dg(us[3], pidx)))) * 0.25
                xt = xt_v[gs]
                x1 = x1_v[gs]
                x2 = x2_v[gs]
                yt = yt_v[gs]
                y1 = y1_v[gs]
                y2 = y2_v[gs]
                t = x1 * y1 + x2 * y2 - xt * yt
                ot = yt + xt * t
                o1 = y1 + x1 * t
                o2 = y2 + x2 * t
                q = o1 * o1 + o2 * o2 - ot * ot + EPS
                rinv = _rsqrt(q)
                arg = jnp.maximum(-t, 1.000001)
                s2 = (arg - 1.0) * (arg + 1.0)
                dist = _log(arg + s2 * _rsqrt(s2))
                sc = w * dist * rinv
                ot_v[gs] = ot * sc
                o1_v[gs] = o1 * sc
                o2_v[gs] = o2 * sc
            wps = [pltpu.async_copy(ot_v, at.at[si], sem2, add=True),
                   pltpu.async_copy(o1_v, a1.at[si], sem2, add=True),
                   pltpu.async_copy(o2_v, a2.at[si], sem2, add=True),
                   pltpu.async_copy(ones_v, ac.at[si], sem2, add=True)]
            for wp in wps:
                wp.wait()
        return carry

    lax.fori_loop(0, BPW, block, 0)
    plsc.subcore_barrier()

    @pl.when(c == 0)
    def _():
        pltpu.sync_copy(at.at[sl], s0t.at[sl])
        pltpu.sync_copy(a1.at[sl], s01.at[sl])
        pltpu.sync_copy(a2.at[sl], s02.at[sl])
        pltpu.sync_copy(ac.at[sl], s0c.at[sl])

    @pl.when(c == 1)
    def _():
        pltpu.sync_copy(at.at[sl], s1t.at[sl])
        pltpu.sync_copy(a1.at[sl], s11.at[sl])
        pltpu.sync_copy(a2.at[sl], s12.at[sl])
        pltpu.sync_copy(ac.at[sl], s1c.at[sl])


@functools.partial(
    pl.kernel,
    mesh=_mesh,
    compiler_params=pltpu.CompilerParams(needs_layout_passes=False),
    out_type=jax.ShapeDtypeStruct((NPAD * 3,), _f32),
    scratch_types=(
        [pltpu.VMEM((512,), _f32)] * 14
        + [pltpu.VMEM((1536,), _f32)]
        + [pltpu.SemaphoreType.DMA]
    ),
)
def _exp(s0t, s01, s02, s0c, s1t, s11, s12, s1c, t0h, t1h, t2h,
         out_h,
         b0t, b01, b02, b0c, b1t, b11, b12, b1c, bvt, bv1, bv2,
         ob0, ob1, ob2, obi, sem):
    c = lax.axis_index("c")
    s = lax.axis_index("s")
    wid = s * 2 + c

    def block(i, carry):
        b = wid * FBW + i

        @pl.when(b < FNB)
        def _():
            base = b * 512
            bs = pl.ds(base, 512)
            cps = [pltpu.async_copy(s0t.at[bs], b0t, sem),
                   pltpu.async_copy(s01.at[bs], b01, sem),
                   pltpu.async_copy(s02.at[bs], b02, sem),
                   pltpu.async_copy(s0c.at[bs], b0c, sem),
                   pltpu.async_copy(s1t.at[bs], b1t, sem),
                   pltpu.async_copy(s11.at[bs], b11, sem),
                   pltpu.async_copy(s12.at[bs], b12, sem),
                   pltpu.async_copy(s1c.at[bs], b1c, sem),
                   pltpu.async_copy(t0h.at[bs], bvt, sem),
                   pltpu.async_copy(t1h.at[bs], bv1, sem),
                   pltpu.async_copy(t2h.at[bs], bv2, sem)]
            for cp in cps:
                cp.wait()
            for g in range(32):
                gs = pl.ds(g * 16, 16)
                t0 = b0t[gs] + b1t[gs]
                t1 = b01[gs] + b11[gs]
                t2 = b02[gs] + b12[gs]
                cnt = b0c[gs] + b1c[gs]
                inv = 1.0 / jnp.maximum(cnt, 1.0)
                pos = cnt > 0.0
                t0 = jnp.where(pos, t0 * inv, 0.0)
                t1 = jnp.where(pos, t1 * inv, 0.0)
                t2 = jnp.where(pos, t2 * inv, 0.0)
                q = t1 * t1 + t2 * t2 - t0 * t0 + EPS
                r2 = _rsqrt(q)
                T = q * r2
                ee = jnp.exp(T)
                ei = 1.0 / ee
                ch = (ee + ei) * 0.5
                sh = (ee - ei) * 0.5
                r0 = ch * bvt[gs] + sh * (t0 * r2)
                r1 = ch * bv1[gs] + sh * (t1 * r2)
                r2v = ch * bv2[gs] + sh * (t2 * r2)
                ob0[gs] = r0
                ob1[gs] = r1
                ob2[gs] = r2v
            iota = lax.iota(_i32, 16)
            for g in range(32):
                o0 = ob0[pl.ds(g * 16, 16)]
                o1 = ob1[pl.ds(g * 16, 16)]
                o2 = ob2[pl.ds(g * 16, 16)]
                for k in range(3):
                    obi[pl.ds(g * 48 + k * 16, 16)] = _int3(o0, o1, o2, k, iota)
            pltpu.sync_copy(obi, out_h.at[pl.ds(base * 3, 1536)])
        return carry

    lax.fori_loop(0, FBW, block, 0)


def kernel(vertices, edges, messages):
    vpf = jnp.pad(vertices, ((0, NPAD - N_NODES), (0, 0))).reshape(-1)
    ef = edges.reshape(-1)
    mf = messages.reshape(-1)
    z = jnp.zeros((RPT,), _f32)
    outs = _agg(vpf, ef, mf, z)
    out = _exp(*outs)
    return out.reshape(NPAD, 3)[:N_NODES]


# speedup vs baseline: 8.0202x; 8.0202x over previous
import functools

import jax
import jax.numpy as jnp
from jax import lax
from jax.experimental import pallas as pl
from jax.experimental.pallas import tpu as pltpu
from jax.experimental.pallas import tpu_sc as plsc

N_NODES = 100000
N_EDGES = 3200000
NPAD = 102400
EPS = 1e-6

NB = N_EDGES // 512
BPW = 196
RPT = NPAD // 16
VCH = RPT // 640
FNB = NPAD // 512
FBW = 7

_mesh = plsc.VectorSubcoreMesh(core_axis_name="c", subcore_axis_name="s")

_f32 = jnp.float32
_i32 = jnp.int32


def _rsqrt(x):
    i = lax.bitcast_convert_type(x, _i32)
    i = jnp.int32(0x5F3759DF) - (i >> 1)
    y = lax.bitcast_convert_type(i, _f32)
    y = y * (1.5 - 0.5 * x * y * y)
    y = y * (1.5 - 0.5 * x * y * y)
    return y


def _log(x):
    bits = lax.bitcast_convert_type(x, _i32)
    e = (bits >> 23) - 127
    m = lax.bitcast_convert_type((bits & 0x7FFFFF) | 0x3F800000, _f32)
    big = m > 1.4142135
    m = jnp.where(big, m * 0.5, m)
    ef = (e + big.astype(_i32)).astype(_f32)
    t = (m - 1.0) / (m + 1.0)
    t2 = t * t
    p = 2.0 * t * (1.0 + t2 * (1.0 / 3.0 + t2 * (0.2 + t2 * (1.0 / 7.0))))
    return ef * 0.6931471805599453 + p


def _dg(v, idx):
    return lax.gather(
        v, idx[:, None],
        lax.GatherDimensionNumbers(offset_dims=(), collapsed_slice_dims=(0,),
                                   start_index_map=(0,)),
        (1,), mode=lax.GatherScatterMode.PROMISE_IN_BOUNDS)


_PART = jax.ShapeDtypeStruct((NPAD,), _f32)


def _deint3(vA, vB, vC, c, iota):
    pos = 3 * iota + c
    idx = pos & 15
    sel = pos >> 4
    return jnp.where(sel == 0, _dg(vA, idx),
                     jnp.where(sel == 1, _dg(vB, idx), _dg(vC, idx)))


def _int3(o0, o1, o2, k, iota):
    pos = 16 * k + iota
    idx = (pos * 43691) >> 17
    cmp = pos - 3 * idx
    return jnp.where(cmp == 0, _dg(o0, idx),
                     jnp.where(cmp == 1, _dg(o1, idx), _dg(o2, idx)))


@functools.partial(
    pl.kernel,
    mesh=_mesh,
    compiler_params=pltpu.CompilerParams(needs_layout_passes=False),
    out_type=[_PART] * 11,
    scratch_types=(
        [pltpu.VMEM_SHARED((NPAD,), _f32)] * 7
        + [pltpu.VMEM((1920,), _f32)]
        + [pltpu.VMEM((640,), _f32)] * 3
        + [pltpu.VMEM((1024,), _i32)]
        + [pltpu.VMEM((512,), _i32)] * 2
        + [pltpu.VMEM((2048,), _f32)]
        + [pltpu.VMEM((512,), _f32)] * 10
        + [pltpu.SemaphoreType.DMA] * 2
    ),
)
def _agg(vp_h, edges_h, msg_h, z_h,
         s0t, s01, s02, s0c, s1t, s11, s12, s1c, t0h, t1h, t2h,
         vt, v1, v2, at, a1, a2, ac,
         vfl, tb0, tb1, tb2, ed_v, si, di, msgv,
         xt_v, x1_v, x2_v, yt_v, y1_v, y2_v, ot_v, o1_v, o2_v, ones_v,
         sem, sem2):
    c = lax.axis_index("c")
    s = lax.axis_index("s")
    wid = s * 2 + c
    lo = s * RPT
    iota = lax.iota(_i32, 16)

    for ch in range(VCH):
        rbase = lo + ch * 640
        pltpu.sync_copy(vp_h.at[pl.ds(rbase * 3, 1920)], vfl)
        for g in range(40):
            vA = vfl[pl.ds(g * 48, 16)]
            vB = vfl[pl.ds(g * 48 + 16, 16)]
            vC = vfl[pl.ds(g * 48 + 32, 16)]
            gs = pl.ds(g * 16, 16)
            tb0[gs] = _deint3(vA, vB, vC, 0, iota)
            tb1[gs] = _deint3(vA, vB, vC, 1, iota)
            tb2[gs] = _deint3(vA, vB, vC, 2, iota)
        cs = pl.ds(rbase, 640)
        pltpu.sync_copy(tb0, vt.at[cs])
        pltpu.sync_copy(tb1, v1.at[cs])
        pltpu.sync_copy(tb2, v2.at[cs])

        @pl.when(c == 0)
        def _():
            pltpu.sync_copy(tb0, t0h.at[cs])
            pltpu.sync_copy(tb1, t1h.at[cs])
            pltpu.sync_copy(tb2, t2h.at[cs])

    sl = pl.ds(lo, RPT)
    pltpu.sync_copy(z_h, at.at[sl])
    pltpu.sync_copy(z_h, a1.at[sl])
    pltpu.sync_copy(z_h, a2.at[sl])
    pltpu.sync_copy(z_h, ac.at[sl])
    plsc.subcore_barrier()

    one16 = jnp.ones((16,), _f32)
    for g in range(32):
        ones_v[pl.ds(g * 16, 16)] = one16

    def block(i, carry):
        b = wid * BPW + i

        @pl.when(b < NB)
        def _():
            pltpu.sync_copy(edges_h.at[pl.ds(b * 1024, 1024)], ed_v)
            pltpu.sync_copy(msg_h.at[pl.ds(b * 2048, 2048)], msgv)
            for g in range(32):
                k, u = divmod(g, 8)
                gs = pl.ds(g * 16, 16)
                si[gs] = ed_v[pl.ds(k * 256 + u * 16, 16)]
                di[gs] = ed_v[pl.ds(k * 256 + 128 + u * 16, 16)]
            cps = [pltpu.async_copy(vt.at[si], xt_v, sem),
                   pltpu.async_copy(v1.at[si], x1_v, sem),
                   pltpu.async_copy(v2.at[si], x2_v, sem),
                   pltpu.async_copy(vt.at[di], yt_v, sem),
                   pltpu.async_copy(v1.at[di], y1_v, sem),
                   pltpu.async_copy(v2.at[di], y2_v, sem)]
            for cp in cps:
                cp.wait()
            for g in range(32):
                gs = pl.ds(g * 16, 16)
                k, u = divmod(g, 8)
                mb = k * 512 + u * 16
                w = (msgv[pl.ds(mb, 16)] + msgv[pl.ds(mb + 128, 16)]
                     + msgv[pl.ds(mb + 256, 16)]
                     + msgv[pl.ds(mb + 384, 16)]) * 0.25
                xt = xt_v[gs]
                x1 = x1_v[gs]
                x2 = x2_v[gs]
                yt = yt_v[gs]
                y1 = y1_v[gs]
                y2 = y2_v[gs]
                t = x1 * y1 + x2 * y2 - xt * yt
                ot = yt + xt * t
                o1 = y1 + x1 * t
                o2 = y2 + x2 * t
                q = o1 * o1 + o2 * o2 - ot * ot + EPS
                rinv = _rsqrt(q)
                arg = jnp.maximum(-t, 1.000001)
                s2 = (arg - 1.0) * (arg + 1.0)
                dist = _log(arg + s2 * _rsqrt(s2))
                sc = w * dist * rinv
                ot_v[gs] = ot * sc
                o1_v[gs] = o1 * sc
                o2_v[gs] = o2 * sc
            wps = [pltpu.async_copy(ot_v, at.at[si], sem2, add=True),
                   pltpu.async_copy(o1_v, a1.at[si], sem2, add=True),
                   pltpu.async_copy(o2_v, a2.at[si], sem2, add=True),
                   pltpu.async_copy(ones_v, ac.at[si], sem2, add=True)]
            for wp in wps:
                wp.wait()
        return carry

    lax.fori_loop(0, BPW, block, 0)
    plsc.subcore_barrier()

    @pl.when(c == 0)
    def _():
        pltpu.sync_copy(at.at[sl], s0t.at[sl])
        pltpu.sync_copy(a1.at[sl], s01.at[sl])
        pltpu.sync_copy(a2.at[sl], s02.at[sl])
        pltpu.sync_copy(ac.at[sl], s0c.at[sl])

    @pl.when(c == 1)
    def _():
        pltpu.sync_copy(at.at[sl], s1t.at[sl])
        pltpu.sync_copy(a1.at[sl], s11.at[sl])
        pltpu.sync_copy(a2.at[sl], s12.at[sl])
        pltpu.sync_copy(ac.at[sl], s1c.at[sl])


@functools.partial(
    pl.kernel,
    mesh=_mesh,
    compiler_params=pltpu.CompilerParams(needs_layout_passes=False),
    out_type=jax.ShapeDtypeStruct((NPAD * 3,), _f32),
    scratch_types=(
        [pltpu.VMEM((512,), _f32)] * 14
        + [pltpu.VMEM((1536,), _f32)]
        + [pltpu.SemaphoreType.DMA]
    ),
)
def _exp(s0t, s01, s02, s0c, s1t, s11, s12, s1c, t0h, t1h, t2h,
         out_h,
         b0t, b01, b02, b0c, b1t, b11, b12, b1c, bvt, bv1, bv2,
         ob0, ob1, ob2, obi, sem):
    c = lax.axis_index("c")
    s = lax.axis_index("s")
    wid = s * 2 + c

    def block(i, carry):
        b = wid * FBW + i

        @pl.when(b < FNB)
        def _():
            base = b * 512
            bs = pl.ds(base, 512)
            cps = [pltpu.async_copy(s0t.at[bs], b0t, sem),
                   pltpu.async_copy(s01.at[bs], b01, sem),
                   pltpu.async_copy(s02.at[bs], b02, sem),
                   pltpu.async_copy(s0c.at[bs], b0c, sem),
                   pltpu.async_copy(s1t.at[bs], b1t, sem),
                   pltpu.async_copy(s11.at[bs], b11, sem),
                   pltpu.async_copy(s12.at[bs], b12, sem),
                   pltpu.async_copy(s1c.at[bs], b1c, sem),
                   pltpu.async_copy(t0h.at[bs], bvt, sem),
                   pltpu.async_copy(t1h.at[bs], bv1, sem),
                   pltpu.async_copy(t2h.at[bs], bv2, sem)]
            for cp in cps:
                cp.wait()
            for g in range(32):
                gs = pl.ds(g * 16, 16)
                t0 = b0t[gs] + b1t[gs]
                t1 = b01[gs] + b11[gs]
                t2 = b02[gs] + b12[gs]
                cnt = b0c[gs] + b1c[gs]
                inv = 1.0 / jnp.maximum(cnt, 1.0)
                pos = cnt > 0.0
                t0 = jnp.where(pos, t0 * inv, 0.0)
                t1 = jnp.where(pos, t1 * inv, 0.0)
                t2 = jnp.where(pos, t2 * inv, 0.0)
                q = t1 * t1 + t2 * t2 - t0 * t0 + EPS
                r2 = _rsqrt(q)
                T = q * r2
                ee = jnp.exp(T)
                ei = 1.0 / ee
                ch = (ee + ei) * 0.5
                sh = (ee - ei) * 0.5
                r0 = ch * bvt[gs] + sh * (t0 * r2)
                r1 = ch * bv1[gs] + sh * (t1 * r2)
                r2v = ch * bv2[gs] + sh * (t2 * r2)
                ob0[gs] = r0
                ob1[gs] = r1
                ob2[gs] = r2v
            iota = lax.iota(_i32, 16)
            for g in range(32):
                o0 = ob0[pl.ds(g * 16, 16)]
                o1 = ob1[pl.ds(g * 16, 16)]
                o2 = ob2[pl.ds(g * 16, 16)]
                for k in range(3):
                    obi[pl.ds(g * 48 + k * 16, 16)] = _int3(o0, o1, o2, k, iota)
            pltpu.sync_copy(obi, out_h.at[pl.ds(base * 3, 1536)])
        return carry

    lax.fori_loop(0, FBW, block, 0)


def kernel(vertices, edges, messages):
    vpf = jnp.pad(vertices, ((0, NPAD - N_NODES), (0, 0))).reshape(-1)
    ef = edges.reshape(N_EDGES // 128, 128, 2).transpose(0, 2, 1).reshape(-1)
    mf = messages.reshape(N_EDGES // 128, 128, 4).transpose(0, 2, 1).reshape(-1)
    z = jnp.zeros((RPT,), _f32)
    outs = _agg(vpf, ef, mf, z)
    out = _exp(*outs)
    return out.reshape(NPAD, 3)[:N_NODES]


# speedup vs baseline: 8.8292x; 1.1009x over previous
import functools

import jax
import jax.numpy as jnp
from jax import lax
from jax.experimental import pallas as pl
from jax.experimental.pallas import tpu as pltpu
from jax.experimental.pallas import tpu_sc as plsc

N_NODES = 100000
N_EDGES = 3200000
NPAD = 102400
EPS = 1e-6

NB = N_EDGES // 512
BPW = 196
RPT = NPAD // 16
VCH = RPT // 640
FNB = NPAD // 512
FBW = 7

_mesh = plsc.VectorSubcoreMesh(core_axis_name="c", subcore_axis_name="s")

_f32 = jnp.float32
_i32 = jnp.int32


def _rsqrt(x):
    i = lax.bitcast_convert_type(x, _i32)
    i = jnp.int32(0x5F3759DF) - (i >> 1)
    y = lax.bitcast_convert_type(i, _f32)
    y = y * (1.5 - 0.5 * x * y * y)
    y = y * (1.5 - 0.5 * x * y * y)
    return y


def _log(x):
    bits = lax.bitcast_convert_type(x, _i32)
    e = (bits >> 23) - 127
    m = lax.bitcast_convert_type((bits & 0x7FFFFF) | 0x3F800000, _f32)
    big = m > 1.4142135
    m = jnp.where(big, m * 0.5, m)
    ef = (e + big.astype(_i32)).astype(_f32)
    t = (m - 1.0) / (m + 1.0)
    t2 = t * t
    p = 2.0 * t * (1.0 + t2 * (1.0 / 3.0 + t2 * (0.2 + t2 * (1.0 / 7.0))))
    return ef * 0.6931471805599453 + p


def _dg(v, idx):
    return lax.gather(
        v, idx[:, None],
        lax.GatherDimensionNumbers(offset_dims=(), collapsed_slice_dims=(0,),
                                   start_index_map=(0,)),
        (1,), mode=lax.GatherScatterMode.PROMISE_IN_BOUNDS)


_PART = jax.ShapeDtypeStruct((NPAD,), _f32)


def _deint3(vA, vB, vC, c, iota):
    pos = 3 * iota + c
    idx = pos & 15
    sel = pos >> 4
    return jnp.where(sel == 0, _dg(vA, idx),
                     jnp.where(sel == 1, _dg(vB, idx), _dg(vC, idx)))


def _int3(o0, o1, o2, k, iota):
    pos = 16 * k + iota
    idx = (pos * 43691) >> 17
    cmp = pos - 3 * idx
    return jnp.where(cmp == 0, _dg(o0, idx),
                     jnp.where(cmp == 1, _dg(o1, idx), _dg(o2, idx)))


@functools.partial(
    pl.kernel,
    mesh=_mesh,
    compiler_params=pltpu.CompilerParams(needs_layout_passes=False),
    out_type=[_PART] * 11,
    scratch_types=(
        [pltpu.VMEM_SHARED((NPAD,), _f32)] * 7
        + [pltpu.VMEM((1920,), _f32)]
        + [pltpu.VMEM((640,), _f32)] * 3
        + [pltpu.VMEM((1024,), _i32)] * 2
        + [pltpu.VMEM((2048,), _f32)] * 2
        + [pltpu.VMEM((512,), _i32)] * 4
        + [pltpu.VMEM((512,), _f32)] * 12
        + [pltpu.VMEM((512,), _f32)] * 6
        + [pltpu.VMEM((512,), _f32)]
        + [pltpu.SemaphoreType.DMA] * 5
    ),
)
def _agg(vp_h, edges_h, msg_h, z_h,
         s0t, s01, s02, s0c, s1t, s11, s12, s1c, t0h, t1h, t2h,
         vt, v1, v2, at, a1, a2, ac,
         vfl, tb0, tb1, tb2, ed0, ed1, mg0, mg1, si0, si1, di0, di1,
         xt0, x10, x20, yt0, y10, y20, xt1, x11, x21, yt1, y11, y21,
         ot0, o10, o20, ot1, o11, o21, ones_v,
         lsem0, lsem1, gsem0, gsem1, ssem):
    c = lax.axis_index("c")
    s = lax.axis_index("s")
    wid = s * 2 + c
    lo = s * RPT
    iota = lax.iota(_i32, 16)

    for ch in range(VCH):
        rbase = lo + ch * 640
        pltpu.sync_copy(vp_h.at[pl.ds(rbase * 3, 1920)], vfl)
        for g in range(40):
            vA = vfl[pl.ds(g * 48, 16)]
            vB = vfl[pl.ds(g * 48 + 16, 16)]
            vC = vfl[pl.ds(g * 48 + 32, 16)]
            gs = pl.ds(g * 16, 16)
            tb0[gs] = _deint3(vA, vB, vC, 0, iota)
            tb1[gs] = _deint3(vA, vB, vC, 1, iota)
            tb2[gs] = _deint3(vA, vB, vC, 2, iota)
        cs = pl.ds(rbase, 640)
        pltpu.sync_copy(tb0, vt.at[cs])
        pltpu.sync_copy(tb1, v1.at[cs])
        pltpu.sync_copy(tb2, v2.at[cs])

        @pl.when(c == 0)
        def _():
            pltpu.sync_copy(tb0, t0h.at[cs])
            pltpu.sync_copy(tb1, t1h.at[cs])
            pltpu.sync_copy(tb2, t2h.at[cs])

    sl = pl.ds(lo, RPT)
    pltpu.sync_copy(z_h, at.at[sl])
    pltpu.sync_copy(z_h, a1.at[sl])
    pltpu.sync_copy(z_h, a2.at[sl])
    pltpu.sync_copy(z_h, ac.at[sl])
    plsc.subcore_barrier()

    one16 = jnp.ones((16,), _f32)
    for g in range(32):
        ones_v[pl.ds(g * 16, 16)] = one16

    eds = [ed0, ed1]
    mgs = [mg0, mg1]
    sis = [si0, si1]
    dis = [di0, di1]
    xts = [xt0, xt1]
    x1s = [x10, x11]
    x2s = [x20, x21]
    yts = [yt0, yt1]
    y1s = [y10, y11]
    y2s = [y20, y21]
    ots = [ot0, ot1]
    o1s = [o10, o11]
    o2s = [o20, o21]
    lsems = [lsem0, lsem1]
    gsems = [gsem0, gsem1]
    nact = jnp.minimum(jnp.int32(BPW), jnp.int32(NB) - wid * BPW)

    def lin_refs(j, pp):
        return [(edges_h.at[pl.ds((wid * BPW + j) * 1024, 1024)], eds[pp]),
                (msg_h.at[pl.ds((wid * BPW + j) * 2048, 2048)], mgs[pp])]

    def g_refs(pp):
        return [(vt.at[sis[pp]], xts[pp]), (v1.at[sis[pp]], x1s[pp]),
                (v2.at[sis[pp]], x2s[pp]), (vt.at[dis[pp]], yts[pp]),
                (v1.at[dis[pp]], y1s[pp]), (v2.at[dis[pp]], y2s[pp])]

    def build_idx(pp):
        for g in range(32):
            k, u = divmod(g, 8)
            gs = pl.ds(g * 16, 16)
            sis[pp][gs] = eds[pp][pl.ds(k * 256 + u * 16, 16)]
            dis[pp][gs] = eds[pp][pl.ds(k * 256 + 128 + u * 16, 16)]

    def compute(pp):
        mgv = mgs[pp]
        for g in range(32):
            gs = pl.ds(g * 16, 16)
            k, u = divmod(g, 8)
            mb = k * 512 + u * 16
            w = (mgv[pl.ds(mb, 16)] + mgv[pl.ds(mb + 128, 16)]
                 + mgv[pl.ds(mb + 256, 16)] + mgv[pl.ds(mb + 384, 16)]) * 0.25
            xt = xts[pp][gs]
            x1 = x1s[pp][gs]
            x2 = x2s[pp][gs]
            yt = yts[pp][gs]
            y1 = y1s[pp][gs]
            y2 = y2s[pp][gs]
            t = x1 * y1 + x2 * y2 - xt * yt
            ot = yt + xt * t
            o1 = y1 + x1 * t
            o2 = y2 + x2 * t
            q = o1 * o1 + o2 * o2 - ot * ot + EPS
            rinv = _rsqrt(q)
            arg = jnp.maximum(-t, 1.000001)
            s2 = (arg - 1.0) * (arg + 1.0)
            dist = _log(arg + s2 * _rsqrt(s2))
            sc = w * dist * rinv
            ots[pp][gs] = ot * sc
            o1s[pp][gs] = o1 * sc
            o2s[pp][gs] = o2 * sc

    def scatter(pp):
        wps = [pltpu.async_copy(ots[pp], at.at[sis[pp]], ssem, add=True),
               pltpu.async_copy(o1s[pp], a1.at[sis[pp]], ssem, add=True),
               pltpu.async_copy(o2s[pp], a2.at[sis[pp]], ssem, add=True),
               pltpu.async_copy(ones_v, ac.at[sis[pp]], ssem, add=True)]
        for wp in wps:
            wp.wait()

    for sr, dr in lin_refs(0, 0):
        pltpu.sync_copy(sr, dr)
    build_idx(0)
    for sr, dr in g_refs(0):
        pltpu.async_copy(sr, dr, gsems[0])
    for sr, dr in lin_refs(1, 1):
        pltpu.async_copy(sr, dr, lsems[1])

    def two_blocks(ii, carry):
        for p in range(2):
            i = ii * 2 + p
            np_ = 1 - p

            @pl.when(i < nact)
            def _():
                for sr, dr in g_refs(p):
                    pltpu.make_async_copy(sr, dr, gsems[p]).wait()

            @pl.when(i + 1 < nact)
            def _():
                for sr, dr in lin_refs(i + 1, np_):
                    pltpu.make_async_copy(sr, dr, lsems[np_]).wait()
                build_idx(np_)
                for sr, dr in g_refs(np_):
                    pltpu.async_copy(sr, dr, gsems[np_])

            @pl.when(i < nact)
            def _():
                compute(p)
                scatter(p)

            @pl.when(i + 2 < nact)
            def _():
                for sr, dr in lin_refs(i + 2, p):
                    pltpu.async_copy(sr, dr, lsems[p])
        return carry

    lax.fori_loop(0, BPW // 2, two_blocks, 0)
    plsc.subcore_barrier()

    @pl.when(c == 0)
    def _():
        pltpu.sync_copy(at.at[sl], s0t.at[sl])
        pltpu.sync_copy(a1.at[sl], s01.at[sl])
        pltpu.sync_copy(a2.at[sl], s02.at[sl])
        pltpu.sync_copy(ac.at[sl], s0c.at[sl])

    @pl.when(c == 1)
    def _():
        pltpu.sync_copy(at.at[sl], s1t.at[sl])
        pltpu.sync_copy(a1.at[sl], s11.at[sl])
        pltpu.sync_copy(a2.at[sl], s12.at[sl])
        pltpu.sync_copy(ac.at[sl], s1c.at[sl])


@functools.partial(
    pl.kernel,
    mesh=_mesh,
    compiler_params=pltpu.CompilerParams(needs_layout_passes=False),
    out_type=jax.ShapeDtypeStruct((NPAD * 3,), _f32),
    scratch_types=(
        [pltpu.VMEM((512,), _f32)] * 14
        + [pltpu.VMEM((1536,), _f32)]
        + [pltpu.SemaphoreType.DMA]
    ),
)
def _exp(s0t, s01, s02, s0c, s1t, s11, s12, s1c, t0h, t1h, t2h,
         out_h,
         b0t, b01, b02, b0c, b1t, b11, b12, b1c, bvt, bv1, bv2,
         ob0, ob1, ob2, obi, sem):
    c = lax.axis_index("c")
    s = lax.axis_index("s")
    wid = s * 2 + c

    def block(i, carry):
        b = wid * FBW + i

        @pl.when(b < FNB)
        def _():
            base = b * 512
            bs = pl.ds(base, 512)
            cps = [pltpu.async_copy(s0t.at[bs], b0t, sem),
                   pltpu.async_copy(s01.at[bs], b01, sem),
                   pltpu.async_copy(s02.at[bs], b02, sem),
                   pltpu.async_copy(s0c.at[bs], b0c, sem),
                   pltpu.async_copy(s1t.at[bs], b1t, sem),
                   pltpu.async_copy(s11.at[bs], b11, sem),
                   pltpu.async_copy(s12.at[bs], b12, sem),
                   pltpu.async_copy(s1c.at[bs], b1c, sem),
                   pltpu.async_copy(t0h.at[bs], bvt, sem),
                   pltpu.async_copy(t1h.at[bs], bv1, sem),
                   pltpu.async_copy(t2h.at[bs], bv2, sem)]
            for cp in cps:
                cp.wait()
            for g in range(32):
                gs = pl.ds(g * 16, 16)
                t0 = b0t[gs] + b1t[gs]
                t1 = b01[gs] + b11[gs]
                t2 = b02[gs] + b12[gs]
                cnt = b0c[gs] + b1c[gs]
                inv = 1.0 / jnp.maximum(cnt, 1.0)
                pos = cnt > 0.0
                t0 = jnp.where(pos, t0 * inv, 0.0)
                t1 = jnp.where(pos, t1 * inv, 0.0)
                t2 = jnp.where(pos, t2 * inv, 0.0)
                q = t1 * t1 + t2 * t2 - t0 * t0 + EPS
                r2 = _rsqrt(q)
                T = q * r2
                ee = jnp.exp(T)
                ei = 1.0 / ee
                ch = (ee + ei) * 0.5
                sh = (ee - ei) * 0.5
                r0 = ch * bvt[gs] + sh * (t0 * r2)
                r1 = ch * bv1[gs] + sh * (t1 * r2)
                r2v = ch * bv2[gs] + sh * (t2 * r2)
                ob0[gs] = r0
                ob1[gs] = r1
                ob2[gs] = r2v
            iota = lax.iota(_i32, 16)
            for g in range(32):
                o0 = ob0[pl.ds(g * 16, 16)]
                o1 = ob1[pl.ds(g * 16, 16)]
                o2 = ob2[pl.ds(g * 16, 16)]
                for k in range(3):
                    obi[pl.ds(g * 48 + k * 16, 16)] = _int3(o0, o1, o2, k, iota)
            pltpu.sync_copy(obi, out_h.at[pl.ds(base * 3, 1536)])
        return carry

    lax.fori_loop(0, FBW, block, 0)


def kernel(vertices, edges, messages):
    vpf = jnp.pad(vertices, ((0, NPAD - N_NODES), (0, 0))).reshape(-1)
    ef = edges.reshape(N_EDGES // 128, 128, 2).transpose(0, 2, 1).reshape(-1)
    mf = messages.reshape(N_EDGES // 128, 128, 4).transpose(0, 2, 1).reshape(-1)
    z = jnp.zeros((RPT,), _f32)
    outs = _agg(vpf, ef, mf, z)
    out = _exp(*outs)
    return out.reshape(NPAD, 3)[:N_NODES]


# speedup vs baseline: 9.1110x; 1.0319x over previous
import functools

import jax
import jax.numpy as jnp
from jax import lax
from jax.experimental import pallas as pl
from jax.experimental.pallas import tpu as pltpu
from jax.experimental.pallas import tpu_sc as plsc

N_NODES = 100000
N_EDGES = 3200000
NPAD = 102400
EPS = 1e-6

NB = N_EDGES // 512
BPW = 196
RPT = NPAD // 16
VCH = RPT // 640
FNB = NPAD // 512
FBW = 7

_mesh = plsc.VectorSubcoreMesh(core_axis_name="c", subcore_axis_name="s")

_f32 = jnp.float32
_i32 = jnp.int32


def _rsqrt(x):
    i = lax.bitcast_convert_type(x, _i32)
    i = jnp.int32(0x5F3759DF) - (i >> 1)
    y = lax.bitcast_convert_type(i, _f32)
    y = y * (1.5 - 0.5 * x * y * y)
    y = y * (1.5 - 0.5 * x * y * y)
    return y


def _rsqrt1(x):
    i = lax.bitcast_convert_type(x, _i32)
    i = jnp.int32(0x5F3759DF) - (i >> 1)
    y = lax.bitcast_convert_type(i, _f32)
    return y * (1.5 - 0.5 * x * y * y)


def _log(x):
    bits = lax.bitcast_convert_type(x, _i32)
    e = (bits >> 23) - 127
    m = lax.bitcast_convert_type((bits & 0x7FFFFF) | 0x3F800000, _f32)
    big = m > 1.4142135
    m = jnp.where(big, m * 0.5, m)
    ef = (e + big.astype(_i32)).astype(_f32)
    t = (m - 1.0) / (m + 1.0)
    t2 = t * t
    p = 2.0 * t * (1.0 + t2 * (1.0 / 3.0 + t2 * (0.2 + t2 * (1.0 / 7.0))))
    return ef * 0.6931471805599453 + p


def _dg(v, idx):
    return lax.gather(
        v, idx[:, None],
        lax.GatherDimensionNumbers(offset_dims=(), collapsed_slice_dims=(0,),
                                   start_index_map=(0,)),
        (1,), mode=lax.GatherScatterMode.PROMISE_IN_BOUNDS)


_PART = jax.ShapeDtypeStruct((NPAD,), _f32)


def _deint3(vA, vB, vC, c, iota):
    pos = 3 * iota + c
    idx = pos & 15
    sel = pos >> 4
    return jnp.where(sel == 0, _dg(vA, idx),
                     jnp.where(sel == 1, _dg(vB, idx), _dg(vC, idx)))


def _int3(o0, o1, o2, k, iota):
    pos = 16 * k + iota
    idx = (pos * 43691) >> 17
    cmp = pos - 3 * idx
    return jnp.where(cmp == 0, _dg(o0, idx),
                     jnp.where(cmp == 1, _dg(o1, idx), _dg(o2, idx)))


@functools.partial(
    pl.kernel,
    mesh=_mesh,
    compiler_params=pltpu.CompilerParams(needs_layout_passes=False),
    out_type=[_PART] * 11,
    scratch_types=(
        [pltpu.VMEM_SHARED((NPAD,), _f32)] * 7
        + [pltpu.VMEM((1920,), _f32)]
        + [pltpu.VMEM((640,), _f32)] * 3
        + [pltpu.VMEM((1024,), _i32)] * 2
        + [pltpu.VMEM((2048,), _f32)] * 2
        + [pltpu.VMEM((512,), _i32)] * 4
        + [pltpu.VMEM((512,), _f32)] * 12
        + [pltpu.VMEM((512,), _f32)] * 6
        + [pltpu.VMEM((512,), _f32)]
        + [pltpu.SemaphoreType.DMA] * 5
    ),
)
def _agg(vp_h, edges_h, msg_h, z_h,
         s0t, s01, s02, s0c, s1t, s11, s12, s1c, t0h, t1h, t2h,
         vt, v1, v2, at, a1, a2, ac,
         vfl, tb0, tb1, tb2, ed0, ed1, mg0, mg1, si0, si1, di0, di1,
         xt0, x10, x20, yt0, y10, y20, xt1, x11, x21, yt1, y11, y21,
         ot0, o10, o20, ot1, o11, o21, ones_v,
         lsem0, lsem1, gsem0, gsem1, ssem):
    c = lax.axis_index("c")
    s = lax.axis_index("s")
    wid = s * 2 + c
    lo = s * RPT
    iota = lax.iota(_i32, 16)

    for ch in range(VCH):
        rbase = lo + ch * 640
        pltpu.sync_copy(vp_h.at[pl.ds(rbase * 3, 1920)], vfl)
        for g in range(40):
            vA = vfl[pl.ds(g * 48, 16)]
            vB = vfl[pl.ds(g * 48 + 16, 16)]
            vC = vfl[pl.ds(g * 48 + 32, 16)]
            gs = pl.ds(g * 16, 16)
            tb0[gs] = _deint3(vA, vB, vC, 0, iota)
            tb1[gs] = _deint3(vA, vB, vC, 1, iota)
            tb2[gs] = _deint3(vA, vB, vC, 2, iota)
        cs = pl.ds(rbase, 640)
        pltpu.sync_copy(tb0, vt.at[cs])
        pltpu.sync_copy(tb1, v1.at[cs])
        pltpu.sync_copy(tb2, v2.at[cs])

        @pl.when(c == 0)
        def _():
            pltpu.sync_copy(tb0, t0h.at[cs])
            pltpu.sync_copy(tb1, t1h.at[cs])
            pltpu.sync_copy(tb2, t2h.at[cs])

    sl = pl.ds(lo, RPT)
    pltpu.sync_copy(z_h, at.at[sl])
    pltpu.sync_copy(z_h, a1.at[sl])
    pltpu.sync_copy(z_h, a2.at[sl])
    pltpu.sync_copy(z_h, ac.at[sl])
    plsc.subcore_barrier()

    one16 = jnp.ones((16,), _f32)
    for g in range(32):
        ones_v[pl.ds(g * 16, 16)] = one16

    eds = [ed0, ed1]
    mgs = [mg0, mg1]
    sis = [si0, si1]
    dis = [di0, di1]
    xts = [xt0, xt1]
    x1s = [x10, x11]
    x2s = [x20, x21]
    yts = [yt0, yt1]
    y1s = [y10, y11]
    y2s = [y20, y21]
    ots = [ot0, ot1]
    o1s = [o10, o11]
    o2s = [o20, o21]
    lsems = [lsem0, lsem1]
    gsems = [gsem0, gsem1]
    nact = jnp.minimum(jnp.int32(BPW), jnp.int32(NB) - wid * BPW)

    def lin_refs(j, pp):
        return [(edges_h.at[pl.ds((wid * BPW + j) * 1024, 1024)], eds[pp]),
                (msg_h.at[pl.ds((wid * BPW + j) * 2048, 2048)], mgs[pp])]

    def g_refs(pp):
        return [(vt.at[sis[pp]], xts[pp]), (v1.at[sis[pp]], x1s[pp]),
                (v2.at[sis[pp]], x2s[pp]), (vt.at[dis[pp]], yts[pp]),
                (v1.at[dis[pp]], y1s[pp]), (v2.at[dis[pp]], y2s[pp])]

    def build_idx(pp):
        for g in range(32):
            k, u = divmod(g, 8)
            gs = pl.ds(g * 16, 16)
            sis[pp][gs] = eds[pp][pl.ds(k * 256 + u * 16, 16)]
            dis[pp][gs] = eds[pp][pl.ds(k * 256 + 128 + u * 16, 16)]

    def compute(pp):
        mgv = mgs[pp]
        for g in range(32):
            gs = pl.ds(g * 16, 16)
            k, u = divmod(g, 8)
            mb = k * 512 + u * 16
            w = (mgv[pl.ds(mb, 16)] + mgv[pl.ds(mb + 128, 16)]
                 + mgv[pl.ds(mb + 256, 16)] + mgv[pl.ds(mb + 384, 16)]) * 0.25
            xt = xts[pp][gs]
            x1 = x1s[pp][gs]
            x2 = x2s[pp][gs]
            yt = yts[pp][gs]
            y1 = y1s[pp][gs]
            y2 = y2s[pp][gs]
            t = x1 * y1 + x2 * y2 - xt * yt
            ot = yt + xt * t
            o1 = y1 + x1 * t
            o2 = y2 + x2 * t
            q = o1 * o1 + o2 * o2 - ot * ot + EPS
            rinv = _rsqrt1(q)
            arg = jnp.maximum(-t, 1.000001)
            s2 = (arg - 1.0) * (arg + 1.0)
            dist = _log(arg + s2 * _rsqrt1(s2))
            sc = w * dist * rinv
            ots[pp][gs] = ot * sc
            o1s[pp][gs] = o1 * sc
            o2s[pp][gs] = o2 * sc

    def scatter(pp):
        wps = [pltpu.async_copy(ots[pp], at.at[sis[pp]], ssem, add=True),
               pltpu.async_copy(o1s[pp], a1.at[sis[pp]], ssem, add=True),
               pltpu.async_copy(o2s[pp], a2.at[sis[pp]], ssem, add=True),
               pltpu.async_copy(ones_v, ac.at[sis[pp]], ssem, add=True)]
        for wp in wps:
            wp.wait()

    for sr, dr in lin_refs(0, 0):
        pltpu.sync_copy(sr, dr)
    build_idx(0)
    for sr, dr in g_refs(0):
        pltpu.async_copy(sr, dr, gsems[0])
    for sr, dr in lin_refs(1, 1):
        pltpu.async_copy(sr, dr, lsems[1])

    def two_blocks(ii, carry):
        for p in range(2):
            i = ii * 2 + p
            np_ = 1 - p

            @pl.when(i < nact)
            def _():
                for sr, dr in g_refs(p):
                    pltpu.make_async_copy(sr, dr, gsems[p]).wait()

            @pl.when(i + 1 < nact)
            def _():
                for sr, dr in lin_refs(i + 1, np_):
                    pltpu.make_async_copy(sr, dr, lsems[np_]).wait()
                build_idx(np_)
                for sr, dr in g_refs(np_):
                    pltpu.async_copy(sr, dr, gsems[np_])

            @pl.when(i < nact)
            def _():
                compute(p)
                scatter(p)

            @pl.when(i + 2 < nact)
            def _():
                for sr, dr in lin_refs(i + 2, p):
                    pltpu.async_copy(sr, dr, lsems[p])
        return carry

    lax.fori_loop(0, BPW // 2, two_blocks, 0)
    plsc.subcore_barrier()

    @pl.when(c == 0)
    def _():
        pltpu.sync_copy(at.at[sl], s0t.at[sl])
        pltpu.sync_copy(a1.at[sl], s01.at[sl])
        pltpu.sync_copy(a2.at[sl], s02.at[sl])
        pltpu.sync_copy(ac.at[sl], s0c.at[sl])

    @pl.when(c == 1)
    def _():
        pltpu.sync_copy(at.at[sl], s1t.at[sl])
        pltpu.sync_copy(a1.at[sl], s11.at[sl])
        pltpu.sync_copy(a2.at[sl], s12.at[sl])
        pltpu.sync_copy(ac.at[sl], s1c.at[sl])


@functools.partial(
    pl.kernel,
    mesh=_mesh,
    compiler_params=pltpu.CompilerParams(needs_layout_passes=False),
    out_type=jax.ShapeDtypeStruct((NPAD * 3,), _f32),
    scratch_types=(
        [pltpu.VMEM((512,), _f32)] * 14
        + [pltpu.VMEM((1536,), _f32)]
        + [pltpu.SemaphoreType.DMA]
    ),
)
def _exp(s0t, s01, s02, s0c, s1t, s11, s12, s1c, t0h, t1h, t2h,
         out_h,
         b0t, b01, b02, b0c, b1t, b11, b12, b1c, bvt, bv1, bv2,
         ob0, ob1, ob2, obi, sem):
    c = lax.axis_index("c")
    s = lax.axis_index("s")
    wid = s * 2 + c

    def block(i, carry):
        b = wid * FBW + i

        @pl.when(b < FNB)
        def _():
            base = b * 512
            bs = pl.ds(base, 512)
            cps = [pltpu.async_copy(s0t.at[bs], b0t, sem),
                   pltpu.async_copy(s01.at[bs], b01, sem),
                   pltpu.async_copy(s02.at[bs], b02, sem),
                   pltpu.async_copy(s0c.at[bs], b0c, sem),
                   pltpu.async_copy(s1t.at[bs], b1t, sem),
                   pltpu.async_copy(s11.at[bs], b11, sem),
                   pltpu.async_copy(s12.at[bs], b12, sem),
                   pltpu.async_copy(s1c.at[bs], b1c, sem),
                   pltpu.async_copy(t0h.at[bs], bvt, sem),
                   pltpu.async_copy(t1h.at[bs], bv1, sem),
                   pltpu.async_copy(t2h.at[bs], bv2, sem)]
            for cp in cps:
                cp.wait()
            for g in range(32):
                gs = pl.ds(g * 16, 16)
                t0 = b0t[gs] + b1t[gs]
                t1 = b01[gs] + b11[gs]
                t2 = b02[gs] + b12[gs]
                cnt = b0c[gs] + b1c[gs]
                inv = 1.0 / jnp.maximum(cnt, 1.0)
                pos = cnt > 0.0
                t0 = jnp.where(pos, t0 * inv, 0.0)
                t1 = jnp.where(pos, t1 * inv, 0.0)
                t2 = jnp.where(pos, t2 * inv, 0.0)
                q = t1 * t1 + t2 * t2 - t0 * t0 + EPS
                r2 = _rsqrt(q)
                T = q * r2
                ee = jnp.exp(T)
                ei = 1.0 / ee
                ch = (ee + ei) * 0.5
                sh = (ee - ei) * 0.5
                r0 = ch * bvt[gs] + sh * (t0 * r2)
                r1 = ch * bv1[gs] + sh * (t1 * r2)
                r2v = ch * bv2[gs] + sh * (t2 * r2)
                ob0[gs] = r0
                ob1[gs] = r1
                ob2[gs] = r2v
            iota = lax.iota(_i32, 16)
            for g in range(32):
                o0 = ob0[pl.ds(g * 16, 16)]
                o1 = ob1[pl.ds(g * 16, 16)]
                o2 = ob2[pl.ds(g * 16, 16)]
                for k in range(3):
                    obi[pl.ds(g * 48 + k * 16, 16)] = _int3(o0, o1, o2, k, iota)
            pltpu.sync_copy(obi, out_h.at[pl.ds(base * 3, 1536)])
        return carry

    lax.fori_loop(0, FBW, block, 0)


def kernel(vertices, edges, messages):
    vpf = jnp.pad(vertices, ((0, NPAD - N_NODES), (0, 0))).reshape(-1)
    ef = edges.reshape(N_EDGES // 128, 128, 2).transpose(0, 2, 1).reshape(-1)
    mf = messages.reshape(N_EDGES // 128, 128, 4).transpose(0, 2, 1).reshape(-1)
    z = jnp.zeros((RPT,), _f32)
    outs = _agg(vpf, ef, mf, z)
    out = _exp(*outs)
    return out.reshape(NPAD, 3)[:N_NODES]


# speedup vs baseline: 12.5923x; 1.3821x over previous
import functools

import jax
import jax.numpy as jnp
from jax import lax
from jax.experimental import pallas as pl
from jax.experimental.pallas import tpu as pltpu
from jax.experimental.pallas import tpu_sc as plsc

N_NODES = 100000
N_EDGES = 3200000
NPAD = 102400
EPS = 1e-6

NB = N_EDGES // 512
BPW = 196
RPT = NPAD // 16
VCH = RPT // 640
FNB = NPAD // 512
FBW = 7

_mesh = plsc.VectorSubcoreMesh(core_axis_name="c", subcore_axis_name="s")

_f32 = jnp.float32
_i32 = jnp.int32


def _rsqrt(x):
    i = lax.bitcast_convert_type(x, _i32)
    i = jnp.int32(0x5F3759DF) - (i >> 1)
    y = lax.bitcast_convert_type(i, _f32)
    y = y * (1.5 - 0.5 * x * y * y)
    y = y * (1.5 - 0.5 * x * y * y)
    return y


def _rsqrt1(x):
    i = lax.bitcast_convert_type(x, _i32)
    i = jnp.int32(0x5F3759DF) - (i >> 1)
    y = lax.bitcast_convert_type(i, _f32)
    return y * (1.5 - 0.5 * x * y * y)


def _log(x):
    bits = lax.bitcast_convert_type(x, _i32)
    e = (bits >> 23) - 127
    m = lax.bitcast_convert_type((bits & 0x7FFFFF) | 0x3F800000, _f32)
    big = m > 1.4142135
    m = jnp.where(big, m * 0.5, m)
    ef = (e + big.astype(_i32)).astype(_f32)
    t = (m - 1.0) / (m + 1.0)
    t2 = t * t
    p = 2.0 * t * (1.0 + t2 * (1.0 / 3.0 + t2 * (0.2 + t2 * (1.0 / 7.0))))
    return ef * 0.6931471805599453 + p


def _dg(v, idx):
    return lax.gather(
        v, idx[:, None],
        lax.GatherDimensionNumbers(offset_dims=(), collapsed_slice_dims=(0,),
                                   start_index_map=(0,)),
        (1,), mode=lax.GatherScatterMode.PROMISE_IN_BOUNDS)


_PART = jax.ShapeDtypeStruct((NPAD,), _f32)


def _deint3(vA, vB, vC, c, iota):
    pos = 3 * iota + c
    idx = pos & 15
    sel = pos >> 4
    return jnp.where(sel == 0, _dg(vA, idx),
                     jnp.where(sel == 1, _dg(vB, idx), _dg(vC, idx)))


def _int3(o0, o1, o2, k, iota):
    pos = 16 * k + iota
    idx = (pos * 43691) >> 17
    cmp = pos - 3 * idx
    return jnp.where(cmp == 0, _dg(o0, idx),
                     jnp.where(cmp == 1, _dg(o1, idx), _dg(o2, idx)))


@functools.partial(
    pl.kernel,
    mesh=_mesh,
    compiler_params=pltpu.CompilerParams(needs_layout_passes=False),
    out_type=[_PART] * 11,
    scratch_types=(
        [pltpu.VMEM_SHARED((NPAD,), _f32)] * 7
        + [pltpu.VMEM((1920,), _f32)]
        + [pltpu.VMEM((640,), _f32)] * 3
        + [pltpu.VMEM((1024,), _i32)] * 2
        + [pltpu.VMEM((2048,), _f32)] * 2
        + [pltpu.VMEM((512,), _i32)] * 6
        + [pltpu.VMEM((512,), _f32)] * 12
        + [pltpu.VMEM((512,), _f32)] * 6
        + [pltpu.VMEM((512,), _f32)]
        + [pltpu.SemaphoreType.DMA] * 6
    ),
)
def _agg(vp_h, edges_h, msg_h, z_h,
         s0t, s01, s02, s0c, s1t, s11, s12, s1c, t0h, t1h, t2h,
         vt, v1, v2, at, a1, a2, ac,
         vfl, tb0, tb1, tb2, ed0, ed1, mg0, mg1,
         si0, si1, di0, di1, ssi0, ssi1,
         xt0, x10, x20, yt0, y10, y20, xt1, x11, x21, yt1, y11, y21,
         ot0, o10, o20, ot1, o11, o21, ones_v,
         lsem0, lsem1, gsem0, gsem1, ssem0, ssem1):
    c = lax.axis_index("c")
    s = lax.axis_index("s")
    wid = s * 2 + c
    lo = s * RPT
    iota = lax.iota(_i32, 16)

    for ch in range(VCH):
        rbase = lo + ch * 640
        pltpu.sync_copy(vp_h.at[pl.ds(rbase * 3, 1920)], vfl)
        for g in range(40):
            vA = vfl[pl.ds(g * 48, 16)]
            vB = vfl[pl.ds(g * 48 + 16, 16)]
            vC = vfl[pl.ds(g * 48 + 32, 16)]
            gs = pl.ds(g * 16, 16)
            tb0[gs] = _deint3(vA, vB, vC, 0, iota)
            tb1[gs] = _deint3(vA, vB, vC, 1, iota)
            tb2[gs] = _deint3(vA, vB, vC, 2, iota)
        cs = pl.ds(rbase, 640)
        pltpu.sync_copy(tb0, vt.at[cs])
        pltpu.sync_copy(tb1, v1.at[cs])
        pltpu.sync_copy(tb2, v2.at[cs])

        @pl.when(c == 0)
        def _():
            pltpu.sync_copy(tb0, t0h.at[cs])
            pltpu.sync_copy(tb1, t1h.at[cs])
            pltpu.sync_copy(tb2, t2h.at[cs])

    sl = pl.ds(lo, RPT)
    pltpu.sync_copy(z_h, at.at[sl])
    pltpu.sync_copy(z_h, a1.at[sl])
    pltpu.sync_copy(z_h, a2.at[sl])
    pltpu.sync_copy(z_h, ac.at[sl])
    plsc.subcore_barrier()

    one16 = jnp.ones((16,), _f32)
    for g in range(32):
        ones_v[pl.ds(g * 16, 16)] = one16

    eds = [ed0, ed1]
    mgs = [mg0, mg1]
    sis = [si0, si1]
    dis = [di0, di1]
    xts = [xt0, xt1]
    x1s = [x10, x11]
    x2s = [x20, x21]
    yts = [yt0, yt1]
    y1s = [y10, y11]
    y2s = [y20, y21]
    ots = [ot0, ot1]
    o1s = [o10, o11]
    o2s = [o20, o21]
    ssis = [ssi0, ssi1]
    lsems = [lsem0, lsem1]
    gsems = [gsem0, gsem1]
    ssems = [ssem0, ssem1]
    nact = jnp.minimum(jnp.int32(BPW), jnp.int32(NB) - wid * BPW)

    def lin_refs(j, pp):
        return [(edges_h.at[pl.ds((wid * BPW + j) * 1024, 1024)], eds[pp]),
                (msg_h.at[pl.ds((wid * BPW + j) * 2048, 2048)], mgs[pp])]

    def g_refs(pp):
        return [(vt.at[sis[pp]], xts[pp]), (v1.at[sis[pp]], x1s[pp]),
                (v2.at[sis[pp]], x2s[pp]), (vt.at[dis[pp]], yts[pp]),
                (v1.at[dis[pp]], y1s[pp]), (v2.at[dis[pp]], y2s[pp])]

    def build_idx(pp):
        for g in range(32):
            k, u = divmod(g, 8)
            gs = pl.ds(g * 16, 16)
            sis[pp][gs] = eds[pp][pl.ds(k * 256 + u * 16, 16)]
            dis[pp][gs] = eds[pp][pl.ds(k * 256 + 128 + u * 16, 16)]

    def compute(pp):
        mgv = mgs[pp]
        for g in range(32):
            gs = pl.ds(g * 16, 16)
            k, u = divmod(g, 8)
            mb = k * 512 + u * 16
            w = (mgv[pl.ds(mb, 16)] + mgv[pl.ds(mb + 128, 16)]
                 + mgv[pl.ds(mb + 256, 16)] + mgv[pl.ds(mb + 384, 16)]) * 0.25
            xt = xts[pp][gs]
            x1 = x1s[pp][gs]
            x2 = x2s[pp][gs]
            yt = yts[pp][gs]
            y1 = y1s[pp][gs]
            y2 = y2s[pp][gs]
            t = x1 * y1 + x2 * y2 - xt * yt
            ot = yt + xt * t
            o1 = y1 + x1 * t
            o2 = y2 + x2 * t
            q = o1 * o1 + o2 * o2 - ot * ot + EPS
            rinv = _rsqrt1(q)
            arg = jnp.maximum(-t, 1.000001)
            s2 = (arg - 1.0) * (arg + 1.0)
            dist = _log(arg + s2 * _rsqrt1(s2))
            sc = w * dist * rinv
            ots[pp][gs] = ot * sc
            o1s[pp][gs] = o1 * sc
            o2s[pp][gs] = o2 * sc

    def s_refs(pp):
        return [(ots[pp], at.at[ssis[pp]]), (o1s[pp], a1.at[ssis[pp]]),
                (o2s[pp], a2.at[ssis[pp]]), (ones_v, ac.at[ssis[pp]])]

    def scatter_fire(pp):
        for g in range(32):
            gs = pl.ds(g * 16, 16)
            ssis[pp][gs] = sis[pp][gs]
        for sr, dr in s_refs(pp):
            pltpu.async_copy(sr, dr, ssems[pp], add=True)

    def scatter_drain(pp):
        for sr, dr in s_refs(pp):
            pltpu.make_async_copy(sr, dr, ssems[pp]).wait()

    for sr, dr in lin_refs(0, 0):
        pltpu.sync_copy(sr, dr)
    build_idx(0)
    for sr, dr in g_refs(0):
        pltpu.async_copy(sr, dr, gsems[0])
    for sr, dr in lin_refs(1, 1):
        pltpu.async_copy(sr, dr, lsems[1])

    def two_blocks(ii, carry):
        for p in range(2):
            i = ii * 2 + p
            np_ = 1 - p

            @pl.when(i < nact)
            def _():
                for sr, dr in g_refs(p):
                    pltpu.make_async_copy(sr, dr, gsems[p]).wait()

            @pl.when(i + 1 < nact)
            def _():
                for sr, dr in lin_refs(i + 1, np_):
                    pltpu.make_async_copy(sr, dr, lsems[np_]).wait()
                build_idx(np_)
                for sr, dr in g_refs(np_):
                    pltpu.async_copy(sr, dr, gsems[np_])

            @pl.when((i >= 2) & (i - 2 < nact))
            def _():
                scatter_drain(p)

            @pl.when(i < nact)
            def _():
                compute(p)
                scatter_fire(p)

            @pl.when(i + 2 < nact)
            def _():
                for sr, dr in lin_refs(i + 2, p):
                    pltpu.async_copy(sr, dr, lsems[p])
        return carry

    lax.fori_loop(0, BPW // 2, two_blocks, 0)

    @pl.when(nact == BPW)
    def _():
        scatter_drain(0)
        scatter_drain(1)

    plsc.subcore_barrier()

    @pl.when(c == 0)
    def _():
        pltpu.sync_copy(at.at[sl], s0t.at[sl])
        pltpu.sync_copy(a1.at[sl], s01.at[sl])
        pltpu.sync_copy(a2.at[sl], s02.at[sl])
        pltpu.sync_copy(ac.at[sl], s0c.at[sl])

    @pl.when(c == 1)
    def _():
        pltpu.sync_copy(at.at[sl], s1t.at[sl])
        pltpu.sync_copy(a1.at[sl], s11.at[sl])
        pltpu.sync_copy(a2.at[sl], s12.at[sl])
        pltpu.sync_copy(ac.at[sl], s1c.at[sl])


@functools.partial(
    pl.kernel,
    mesh=_mesh,
    compiler_params=pltpu.CompilerParams(needs_layout_passes=False),
    out_type=jax.ShapeDtypeStruct((NPAD * 3,), _f32),
    scratch_types=(
        [pltpu.VMEM((512,), _f32)] * 14
        + [pltpu.VMEM((1536,), _f32)]
        + [pltpu.SemaphoreType.DMA]
    ),
)
def _exp(s0t, s01, s02, s0c, s1t, s11, s12, s1c, t0h, t1h, t2h,
         out_h,
         b0t, b01, b02, b0c, b1t, b11, b12, b1c, bvt, bv1, bv2,
         ob0, ob1, ob2, obi, sem):
    c = lax.axis_index("c")
    s = lax.axis_index("s")
    wid = s * 2 + c

    def block(i, carry):
        b = wid * FBW + i

        @pl.when(b < FNB)
        def _():
            base = b * 512
            bs = pl.ds(base, 512)
            cps = [pltpu.async_copy(s0t.at[bs], b0t, sem),
                   pltpu.async_copy(s01.at[bs], b01, sem),
                   pltpu.async_copy(s02.at[bs], b02, sem),
                   pltpu.async_copy(s0c.at[bs], b0c, sem),
                   pltpu.async_copy(s1t.at[bs], b1t, sem),
                   pltpu.async_copy(s11.at[bs], b11, sem),
                   pltpu.async_copy(s12.at[bs], b12, sem),
                   pltpu.async_copy(s1c.at[bs], b1c, sem),
                   pltpu.async_copy(t0h.at[bs], bvt, sem),
                   pltpu.async_copy(t1h.at[bs], bv1, sem),
                   pltpu.async_copy(t2h.at[bs], bv2, sem)]
            for cp in cps:
                cp.wait()
            for g in range(32):
                gs = pl.ds(g * 16, 16)
                t0 = b0t[gs] + b1t[gs]
                t1 = b01[gs] + b11[gs]
                t2 = b02[gs] + b12[gs]
                cnt = b0c[gs] + b1c[gs]
                inv = 1.0 / jnp.maximum(cnt, 1.0)
                pos = cnt > 0.0
                t0 = jnp.where(pos, t0 * inv, 0.0)
                t1 = jnp.where(pos, t1 * inv, 0.0)
                t2 = jnp.where(pos, t2 * inv, 0.0)
                q = t1 * t1 + t2 * t2 - t0 * t0 + EPS
                r2 = _rsqrt(q)
                T = q * r2
                ee = jnp.exp(T)
                ei = 1.0 / ee
                ch = (ee + ei) * 0.5
                sh = (ee - ei) * 0.5
                r0 = ch * bvt[gs] + sh * (t0 * r2)
                r1 = ch * bv1[gs] + sh * (t1 * r2)
                r2v = ch * bv2[gs] + sh * (t2 * r2)
                ob0[gs] = r0
                ob1[gs] = r1
                ob2[gs] = r2v
            iota = lax.iota(_i32, 16)
            for g in range(32):
                o0 = ob0[pl.ds(g * 16, 16)]
                o1 = ob1[pl.ds(g * 16, 16)]
                o2 = ob2[pl.ds(g * 16, 16)]
                for k in range(3):
                    obi[pl.ds(g * 48 + k * 16, 16)] = _int3(o0, o1, o2, k, iota)
            pltpu.sync_copy(obi, out_h.at[pl.ds(base * 3, 1536)])
        return carry

    lax.fori_loop(0, FBW, block, 0)


def kernel(vertices, edges, messages):
    vpf = jnp.pad(vertices, ((0, NPAD - N_NODES), (0, 0))).reshape(-1)
    ef = edges.reshape(N_EDGES // 128, 128, 2).transpose(0, 2, 1).reshape(-1)
    mf = messages.reshape(N_EDGES // 128, 128, 4).transpose(0, 2, 1).reshape(-1)
    z = jnp.zeros((RPT,), _f32)
    outs = _agg(vpf, ef, mf, z)
    out = _exp(*outs)
    return out.reshape(NPAD, 3)[:N_NODES]
